# Initial kernel scaffold; baseline (speedup 1.0000x reference)
import jax
import jax.numpy as jnp
from jax.experimental import pallas as pl


def kernel(x, edge_index, batch, pos_embedding):
    raise NotImplementedError("write your pallas kernel here")


# speedup vs baseline: 1.8360x; 1.8360x over previous
import functools
import math

import jax
import jax.numpy as jnp
from jax import lax
from jax.experimental import pallas as pl
from jax.experimental.pallas import tpu as pltpu
from jax.experimental.pallas import tpu_sc as plsc

_LANES = 16


def _make_degree_kernel(n_nodes: int, n_edges: int):
    info = plsc.get_sparse_core_info()
    num_workers = info.num_cores * info.num_subcores
    assert n_edges % num_workers == 0
    epw = n_edges // num_workers
    assert epw % _LANES == 0 and n_nodes % _LANES == 0
    assert epw % 8 == 0

    mesh = plsc.VectorSubcoreMesh(core_axis_name="c", subcore_axis_name="s")

    @functools.partial(
        pl.kernel,
        mesh=mesh,
        out_type=jax.ShapeDtypeStruct((num_workers, n_nodes), jnp.float32),
        scratch_types=[
            pltpu.VMEM((epw,), jnp.int32),
            pltpu.VMEM((n_nodes,), jnp.float32),
        ],
        compiler_params=pltpu.CompilerParams(needs_layout_passes=False),
    )
    def degree_kernel(row_hbm, out_hbm, idx_v, hist_v):
        c = lax.axis_index("c")
        s = lax.axis_index("s")
        wid = s * info.num_cores + c
        base = wid * epw

        pltpu.sync_copy(row_hbm.at[pl.ds(base, epw)], idx_v)

        zeros = jnp.zeros((_LANES,), jnp.float32)

        def zero_body(i, carry):
            hist_v[pl.ds(i * _LANES, _LANES)] = zeros
            return carry

        lax.fori_loop(0, n_nodes // _LANES, zero_body, 0)

        ones = jnp.ones((_LANES,), jnp.float32)

        def scatter_body(i, carry):
            iv = idx_v[pl.ds(i * _LANES, _LANES)]
            plsc.addupdate_scatter(hist_v, [iv], ones)
            return carry

        lax.fori_loop(0, epw // _LANES, scatter_body, 0)

        pltpu.sync_copy(hist_v, out_hbm.at[wid])

    return degree_kernel


def _pe_body(n_freq, x_ref, p_ref, emb_ref, o_ref):
    deg = jnp.sum(p_ref[...], axis=1, keepdims=True)
    dn = deg / jnp.max(deg)
    acc = x_ref[...]
    for f in range(1, n_freq):
        acc = acc + jnp.sin(dn * (f * math.pi)) * emb_ref[f : f + 1, :]
    o_ref[...] = acc


def kernel(x, edge_index, batch, pos_embedding):
    del batch
    n_nodes, d = x.shape
    n_freq = pos_embedding.shape[0]
    row = edge_index[0]

    partial = _make_degree_kernel(n_nodes, row.shape[0])(row)
    partial_t = partial.T

    out = pl.pallas_call(
        functools.partial(_pe_body, n_freq),
        out_shape=jax.ShapeDtypeStruct((n_nodes, d), jnp.float32),
    )(x, partial_t, pos_embedding)
    return out
